# baseline (speedup 1.0000x reference)
import jax
import jax.numpy as jnp
from jax.experimental import pallas as pl
from jax.experimental.pallas import tpu as pltpu

_N_EXP = 8
_D_IN = 1024
_D_HID = 512


def _top2(x, gate_w):
    logits = jax.lax.dot_general(
        x, gate_w, (((1,), (1,)), ((), ())), preferred_element_type=jnp.float32
    )
    m = jnp.max(logits, axis=-1, keepdims=True)
    p = jnp.exp(logits - m)
    s = p / jnp.sum(p, axis=-1, keepdims=True)
    lane = jax.lax.broadcasted_iota(jnp.int32, s.shape, 1)
    m1 = jnp.max(s, axis=-1, keepdims=True)
    i1 = jnp.min(jnp.where(s >= m1, lane, _N_EXP), axis=-1, keepdims=True)
    s2 = jnp.where(lane == i1, -jnp.inf, s)
    m2 = jnp.max(s2, axis=-1, keepdims=True)
    i2 = jnp.min(jnp.where(s2 >= m2, lane, _N_EXP), axis=-1, keepdims=True)
    return m1, i1.astype(jnp.float32), m2, i2.astype(jnp.float32)


def _swiglu(xb, w1, w2, scale):
    h = jax.lax.dot_general(
        xb, w1, (((1,), (1,)), ((), ())), preferred_element_type=jnp.float32
    )
    y = h[:, :_D_HID]
    g = h[:, _D_HID:]
    act = y * (g * jax.lax.logistic(g))
    if scale is not None:
        act = act * scale
    return jax.lax.dot_general(
        act, w2, (((1,), (1,)), ((), ())),
        preferred_element_type=jnp.float32,
    )


def _moe_body(x_ref, gate_ref, w1_ref, w2_ref, sw1_ref, sw2_ref, z_ref,
              m1_s, i1_s, m2_s, i2_s):
    e = pl.program_id(0)
    x = x_ref[...]

    @pl.when(e == 0)
    def _gate():
        m1, i1, m2, i2 = _top2(x, gate_ref[...])
        m1_s[...] = m1
        i1_s[...] = i1
        m2_s[...] = m2
        i2_s[...] = i2

    @pl.when(e < _N_EXP)
    def _routed():
        ef = e.astype(jnp.float32)
        half = x.shape[0] // 2
        for hh in range(2):
            sl = pl.ds(hh * half, half)
            w_e = (
                jnp.where(i1_s[sl, :] == ef, m1_s[sl, :], 0.0)
                + jnp.where(i2_s[sl, :] == ef, m2_s[sl, :], 0.0)
            )
            contrib = _swiglu(x[hh * half:(hh + 1) * half, :],
                              w1_ref[0], w2_ref[0], w_e)

            @pl.when(e == 0)
            def _():
                z_ref[sl, :] = contrib

            @pl.when(e > 0)
            def _():
                z_ref[sl, :] += contrib

    @pl.when(e == _N_EXP)
    def _shared():
        half = x.shape[0] // 2
        for hh in range(2):
            sl = pl.ds(hh * half, half)
            z_ref[sl, :] += _swiglu(x[hh * half:(hh + 1) * half, :],
                                    sw1_ref[...], sw2_ref[...], None)


def _moe(xf, gate_w, expert_fc1, expert_fc2, shared_fc1, shared_fc2, interpret=False):
    t = xf.shape[0]
    last = _N_EXP - 1
    return pl.pallas_call(
        _moe_body,
        grid=(9,),
        in_specs=[
            pl.BlockSpec((t, _D_IN), lambda e: (0, 0)),
            pl.BlockSpec((_N_EXP, _D_IN), lambda e: (0, 0)),
            pl.BlockSpec((1, 2 * _D_HID, _D_IN), lambda e: (jnp.minimum(e, last), 0, 0)),
            pl.BlockSpec((1, _D_IN, _D_HID), lambda e: (jnp.minimum(e, last), 0, 0)),
            pl.BlockSpec((2 * _D_HID, _D_IN), lambda e: (0, 0)),
            pl.BlockSpec((_D_IN, _D_HID), lambda e: (0, 0)),
        ],
        out_specs=pl.BlockSpec((t, _D_IN), lambda e: (0, 0)),
        out_shape=jax.ShapeDtypeStruct((t, _D_IN), jnp.float32),
        scratch_shapes=[pltpu.VMEM((t, 1), jnp.float32) for _ in range(4)],
        compiler_params=pltpu.CompilerParams(
            dimension_semantics=("arbitrary",),
        ),
        interpret=interpret,
    )(xf, gate_w, expert_fc1, expert_fc2, shared_fc1, shared_fc2)


@jax.jit
def kernel(x, gate_w, expert_fc1, expert_fc2, shared_fc1, shared_fc2):
    xf = x.reshape(-1, _D_IN)
    z = _moe(xf, gate_w, expert_fc1, expert_fc2, shared_fc1, shared_fc2)
    return z.reshape(x.shape)


# speedup vs baseline: 1.0065x; 1.0065x over previous
import jax
import jax.numpy as jnp
from jax.experimental import pallas as pl
from jax.experimental.pallas import tpu as pltpu

_N_EXP = 8
_D_IN = 1024
_D_HID = 512


def _top2(x, gate_w):
    logits = jax.lax.dot_general(
        x, gate_w, (((1,), (1,)), ((), ())), preferred_element_type=jnp.float32
    )
    m = jnp.max(logits, axis=-1, keepdims=True)
    p = jnp.exp(logits - m)
    s = p / jnp.sum(p, axis=-1, keepdims=True)
    lane = jax.lax.broadcasted_iota(jnp.int32, s.shape, 1)
    m1 = jnp.max(s, axis=-1, keepdims=True)
    i1 = jnp.min(jnp.where(s >= m1, lane, _N_EXP), axis=-1, keepdims=True)
    s2 = jnp.where(lane == i1, -jnp.inf, s)
    m2 = jnp.max(s2, axis=-1, keepdims=True)
    i2 = jnp.min(jnp.where(s2 >= m2, lane, _N_EXP), axis=-1, keepdims=True)
    return m1, i1.astype(jnp.float32), m2, i2.astype(jnp.float32)


def _swiglu(xb, w1, w2, scale):
    h = jax.lax.dot_general(
        xb, w1, (((1,), (1,)), ((), ())), preferred_element_type=jnp.float32
    )
    y = h[:, :_D_HID]
    g = h[:, _D_HID:]
    act = y * (g * jax.lax.logistic(g))
    if scale is not None:
        act = act * scale
    return jax.lax.dot_general(
        act, w2, (((1,), (1,)), ((), ())),
        preferred_element_type=jnp.float32,
    )


def _moe_body(x_ref, gate_ref, w1_ref, w2_ref, sw1_ref, sw2_ref, z_ref,
              m1_s, i1_s, m2_s, i2_s):
    e = pl.program_id(0)
    x = x_ref[...]

    @pl.when(e == 0)
    def _gate():
        m1, i1, m2, i2 = _top2(x, gate_ref[...])
        m1_s[...] = m1
        i1_s[...] = i1
        m2_s[...] = m2
        i2_s[...] = i2

    @pl.when(e < _N_EXP)
    def _routed():
        ef = e.astype(jnp.float32)
        w_e = (
            jnp.where(i1_s[...] == ef, m1_s[...], 0.0)
            + jnp.where(i2_s[...] == ef, m2_s[...], 0.0)
        )
        contrib = _swiglu(x, w1_ref[0], w2_ref[0], w_e)

        @pl.when(e == 0)
        def _():
            z_ref[...] = contrib

        @pl.when(e > 0)
        def _():
            z_ref[...] += contrib

    @pl.when(e == _N_EXP)
    def _shared():
        z_ref[...] += _swiglu(x, sw1_ref[...], sw2_ref[...], None)


def _moe(xf, gate_w, expert_fc1, expert_fc2, shared_fc1, shared_fc2, interpret=False):
    t = xf.shape[0]
    last = _N_EXP - 1
    return pl.pallas_call(
        _moe_body,
        grid=(9,),
        in_specs=[
            pl.BlockSpec((t, _D_IN), lambda e: (0, 0)),
            pl.BlockSpec((_N_EXP, _D_IN), lambda e: (0, 0)),
            pl.BlockSpec((1, 2 * _D_HID, _D_IN), lambda e: (jnp.minimum(e, last), 0, 0)),
            pl.BlockSpec((1, _D_IN, _D_HID), lambda e: (jnp.minimum(e, last), 0, 0)),
            pl.BlockSpec((2 * _D_HID, _D_IN), lambda e: (0, 0)),
            pl.BlockSpec((_D_IN, _D_HID), lambda e: (0, 0)),
        ],
        out_specs=pl.BlockSpec((t, _D_IN), lambda e: (0, 0)),
        out_shape=jax.ShapeDtypeStruct((t, _D_IN), jnp.float32),
        scratch_shapes=[pltpu.VMEM((t, 1), jnp.float32) for _ in range(4)],
        compiler_params=pltpu.CompilerParams(
            dimension_semantics=("arbitrary",),
        ),
        interpret=interpret,
    )(xf, gate_w, expert_fc1, expert_fc2, shared_fc1, shared_fc2)


@jax.jit
def kernel(x, gate_w, expert_fc1, expert_fc2, shared_fc1, shared_fc2):
    xf = x.reshape(-1, _D_IN)
    z = _moe(xf, gate_w, expert_fc1, expert_fc2, shared_fc1, shared_fc2)
    return z.reshape(x.shape)
